# Initial kernel scaffold; baseline (speedup 1.0000x reference)
import jax
import jax.numpy as jnp
from jax.experimental import pallas as pl


def kernel(x, edge_index, W1, b1, W2, b2, W3, b3):
    raise NotImplementedError("write your pallas kernel here")


# speedup vs baseline: 8.7138x; 8.7138x over previous
import jax
import jax.numpy as jnp
from jax import lax
from jax.experimental import pallas as pl
from jax.experimental.pallas import tpu as pltpu
from jax.experimental.pallas import tpu_sc as plsc

N = 10000
D = 128
E = 320000

NC = 2
NS = 16
NW = NC * NS
K = 128
NCHUNK = 80
NCB = 40
NPH = NCHUNK // NCB
EPW = K * NCHUNK
EPAD = NW * EPW
NPAD = 10240
RPS = NPAD // NS
RB = 640
GRID = NPAD // RB


def _mesh():
    return plsc.VectorSubcoreMesh(
        core_axis_name="c", subcore_axis_name="s", num_cores=NC, num_subcores=NS
    )


def _sc_deg_body(dst_hbm, out_hbm, dst_v, deg_v):
    c = lax.axis_index("c")
    s = lax.axis_index("s")
    wid = s * NC + c
    pltpu.sync_copy(dst_hbm.at[wid], dst_v)
    zeros16 = jnp.zeros((16,), jnp.float32)
    ones16 = jnp.ones((16,), jnp.float32)

    def zero_body(i, _):
        deg_v[pl.ds(i * 16, 16)] = zeros16
        return 0

    lax.fori_loop(0, NPAD // 16, zero_body, 0)

    def scat_body(i, _):
        d16 = dst_v[pl.ds(i * 16, 16)]
        plsc.addupdate_scatter(deg_v, [d16], ones16)
        return 0

    lax.fori_loop(0, EPW // 16, scat_body, 0)
    pltpu.sync_copy(deg_v, out_hbm.at[wid])


def _sc_deg(dst2d):
    return pl.kernel(
        _sc_deg_body,
        out_type=jax.ShapeDtypeStruct((NW, NPAD), jnp.float32),
        mesh=_mesh(),
        scratch_types=[
            pltpu.VMEM((EPW,), jnp.int32),
            pltpu.VMEM((NPAD,), jnp.float32),
        ],
        compiler_params=pltpu.CompilerParams(needs_layout_passes=False),
    )(dst2d)


def _sc_agg_body(h_hbm, src_hbm, dst_hbm, zeros_hbm, out_hbm,
                 src_v, dst_v, buf0, buf1, acc_sh, sem0, sem1):
    c = lax.axis_index("c")
    s = lax.axis_index("s")
    wid = s * NC + c
    bufs = (buf0, buf1)
    sems = (sem0, sem1)

    pltpu.sync_copy(zeros_hbm, acc_sh.at[pl.ds(s * RPS, RPS)])
    plsc.subcore_barrier()

    def phase(p, _):
        pltpu.sync_copy(src_hbm.at[wid, pl.ds(p * NCB, NCB)], src_v)
        pltpu.sync_copy(dst_hbm.at[wid, pl.ds(p * NCB, NCB)], dst_v)
        pltpu.async_copy(h_hbm.at[src_v.at[0]], buf0, sem0)

        def body(i, _):
            for b in range(2):
                cur = 2 * i + b
                nxt = cur + 1

                @pl.when(nxt < NCB)
                def _():
                    pltpu.async_copy(h_hbm.at[src_v.at[nxt]], bufs[1 - b],
                                     sems[1 - b])

                pltpu.make_async_copy(h_hbm.at[src_v.at[cur]], bufs[b],
                                      sems[b]).wait()
                pltpu.sync_copy(bufs[b], acc_sh.at[dst_v.at[cur]], add=True)
            return 0

        lax.fori_loop(0, NCB // 2, body, 0)
        return 0

    lax.fori_loop(0, NPH, phase, 0)
    plsc.subcore_barrier()
    pltpu.sync_copy(acc_sh.at[pl.ds(s * RPS, RPS)],
                    out_hbm.at[c, pl.ds(s * RPS, RPS)])


def _sc_agg(h, src3, dst3, zeros_blk):
    return pl.kernel(
        _sc_agg_body,
        out_type=jax.ShapeDtypeStruct((NC, NPAD, D), jnp.float32),
        mesh=_mesh(),
        scratch_types=[
            pltpu.VMEM((NCB, K), jnp.int32),
            pltpu.VMEM((NCB, K), jnp.int32),
            pltpu.VMEM((K, D), jnp.float32),
            pltpu.VMEM((K, D), jnp.float32),
            pltpu.VMEM_SHARED((NPAD, D), jnp.float32),
            pltpu.SemaphoreType.DMA,
            pltpu.SemaphoreType.DMA,
        ],
    )(h, src3, dst3, zeros_blk)


def _tc_prep_body(x_ref, w_ref, degp_ref, dinv_ref, h_ref):
    deg = jnp.sum(degp_ref[...], axis=0) + 1.0
    dinv = lax.rsqrt(deg)
    dinv_ref[...] = dinv[:, None]
    h = jnp.dot(x_ref[...], w_ref[...], preferred_element_type=jnp.float32)
    h_ref[...] = dinv[:, None] * h


def _tc_prep(x_p, w1, deg_parts):
    return pl.pallas_call(
        _tc_prep_body,
        grid=(GRID,),
        in_specs=[
            pl.BlockSpec((RB, D), lambda i: (i, 0)),
            pl.BlockSpec((D, D), lambda i: (0, 0)),
            pl.BlockSpec((NW, RB), lambda i: (0, i)),
        ],
        out_specs=[
            pl.BlockSpec((RB, 1), lambda i: (i, 0)),
            pl.BlockSpec((RB, D), lambda i: (i, 0)),
        ],
        out_shape=[
            jax.ShapeDtypeStruct((NPAD, 1), jnp.float32),
            jax.ShapeDtypeStruct((NPAD, D), jnp.float32),
        ],
    )(x_p, w1, deg_parts)


def _tc_mid_body(p_ref, hprev_ref, dinv_ref, b_ref, w_ref, hn_ref):
    dinv = dinv_ref[...]
    o = dinv * (p_ref[0] + p_ref[1] + hprev_ref[...]) + b_ref[...]
    xn = jnp.maximum(o, 0.0)
    hn = jnp.dot(xn, w_ref[...], preferred_element_type=jnp.float32)
    hn_ref[...] = dinv * hn


def _tc_mid(p, hprev, dinv, b2d, w_next):
    return pl.pallas_call(
        _tc_mid_body,
        grid=(GRID,),
        in_specs=[
            pl.BlockSpec((NC, RB, D), lambda i: (0, i, 0)),
            pl.BlockSpec((RB, D), lambda i: (i, 0)),
            pl.BlockSpec((RB, 1), lambda i: (i, 0)),
            pl.BlockSpec((1, D), lambda i: (0, 0)),
            pl.BlockSpec((D, D), lambda i: (0, 0)),
        ],
        out_specs=pl.BlockSpec((RB, D), lambda i: (i, 0)),
        out_shape=jax.ShapeDtypeStruct((NPAD, D), jnp.float32),
    )(p, hprev, dinv, b2d, w_next)


def _tc_final_body(p_ref, hprev_ref, dinv_ref, b_ref, out_ref):
    out_ref[...] = (dinv_ref[...] * (p_ref[0] + p_ref[1] + hprev_ref[...])
                    + b_ref[...])


def _tc_final(p, hprev, dinv, b2d):
    return pl.pallas_call(
        _tc_final_body,
        grid=(GRID,),
        in_specs=[
            pl.BlockSpec((NC, RB, D), lambda i: (0, i, 0)),
            pl.BlockSpec((RB, D), lambda i: (i, 0)),
            pl.BlockSpec((RB, 1), lambda i: (i, 0)),
            pl.BlockSpec((1, D), lambda i: (0, 0)),
        ],
        out_specs=pl.BlockSpec((RB, D), lambda i: (i, 0)),
        out_shape=jax.ShapeDtypeStruct((NPAD, D), jnp.float32),
    )(p, hprev, dinv, b2d)


def kernel(x, edge_index, W1, b1, W2, b2, W3, b3):
    src = edge_index[0].astype(jnp.int32)
    dst = edge_index[1].astype(jnp.int32)
    pad = jnp.full((EPAD - E,), N, dtype=jnp.int32)
    src3 = jnp.concatenate([src, pad]).reshape(NW, NCHUNK, K)
    dst3 = jnp.concatenate([dst, pad]).reshape(NW, NCHUNK, K)
    dst2d = dst3.reshape(NW, EPW)
    x_p = jnp.concatenate([x, jnp.zeros((NPAD - N, D), jnp.float32)], axis=0)
    zeros_blk = jnp.zeros((RPS, D), jnp.float32)

    deg_parts = _sc_deg(dst2d)
    dinv, h1 = _tc_prep(x_p, W1, deg_parts)
    b1_2, b2_2, b3_2 = b1[None, :], b2[None, :], b3[None, :]

    p1 = _sc_agg(h1, src3, dst3, zeros_blk)
    h2 = _tc_mid(p1, h1, dinv, b1_2, W2)
    p2 = _sc_agg(h2, src3, dst3, zeros_blk)
    h3 = _tc_mid(p2, h2, dinv, b2_2, W3)
    p3 = _sc_agg(h3, src3, dst3, zeros_blk)
    out = _tc_final(p3, h3, dinv, b3_2)
    return out[:N]


# speedup vs baseline: 8.9997x; 1.0328x over previous
import jax
import jax.numpy as jnp
from jax import lax
from jax.experimental import pallas as pl
from jax.experimental.pallas import tpu as pltpu
from jax.experimental.pallas import tpu_sc as plsc

N = 10000
D = 128
E = 320000

NC = 2
NS = 16
NW = NC * NS
K = 64
EPW = 10240
NCHUNK = EPW // K
NCB = 40
NPH = NCHUNK // NCB
NBUF = 4
AHEAD = 2
EPAD = NW * EPW
NPAD = 10240
RPS = NPAD // NS
RB = 640
GRID = NPAD // RB


def _mesh():
    return plsc.VectorSubcoreMesh(
        core_axis_name="c", subcore_axis_name="s", num_cores=NC, num_subcores=NS
    )


def _sc_deg_body(dst_hbm, out_hbm, dst_v, deg_v):
    c = lax.axis_index("c")
    s = lax.axis_index("s")
    wid = s * NC + c
    pltpu.sync_copy(dst_hbm.at[wid], dst_v)
    zeros16 = jnp.zeros((16,), jnp.float32)
    ones16 = jnp.ones((16,), jnp.float32)

    def zero_body(i, _):
        deg_v[pl.ds(i * 16, 16)] = zeros16
        return 0

    lax.fori_loop(0, NPAD // 16, zero_body, 0)

    def scat_body(i, _):
        d16 = dst_v[pl.ds(i * 16, 16)]
        plsc.addupdate_scatter(deg_v, [d16], ones16)
        return 0

    lax.fori_loop(0, EPW // 16, scat_body, 0)
    pltpu.sync_copy(deg_v, out_hbm.at[wid])


def _sc_deg(dst2d):
    return pl.kernel(
        _sc_deg_body,
        out_type=jax.ShapeDtypeStruct((NW, NPAD), jnp.float32),
        mesh=_mesh(),
        scratch_types=[
            pltpu.VMEM((EPW,), jnp.int32),
            pltpu.VMEM((NPAD,), jnp.float32),
        ],
        compiler_params=pltpu.CompilerParams(needs_layout_passes=False),
    )(dst2d)


def _sc_agg_body(h_hbm, src_hbm, dst_hbm, zeros_hbm, out_hbm,
                 src_v, dst_v, *rest):
    bufs = rest[0:NBUF]
    gsem = rest[NBUF:2 * NBUF]
    ssem = rest[2 * NBUF:3 * NBUF]
    acc_sh = rest[3 * NBUF]
    c = lax.axis_index("c")
    s = lax.axis_index("s")
    wid = s * NC + c

    pltpu.sync_copy(zeros_hbm, acc_sh.at[pl.ds(s * RPS, RPS)])
    plsc.subcore_barrier()

    def phase(p, _):
        pltpu.sync_copy(src_hbm.at[wid, pl.ds(p * NCB, NCB)], src_v)
        pltpu.sync_copy(dst_hbm.at[wid, pl.ds(p * NCB, NCB)], dst_v)

        for u in range(AHEAD):
            pltpu.async_copy(h_hbm.at[src_v.at[u]], bufs[u], gsem[u])

        def inner(i, _):
            for u in range(NBUF):
                m = i * NBUF + u
                pltpu.make_async_copy(h_hbm.at[src_v.at[m]], bufs[u],
                                      gsem[u]).wait()
                pltpu.async_copy(bufs[u], acc_sh.at[dst_v.at[m]], ssem[u],
                                 add=True)
                nslot = (u + AHEAD) % NBUF

                @pl.when(m + AHEAD < NCB)
                def _():
                    @pl.when(m >= NBUF - AHEAD)
                    def _():
                        pltpu.make_async_copy(
                            bufs[nslot], acc_sh.at[dst_v.at[0]],
                            ssem[nslot]).wait()

                    pltpu.async_copy(h_hbm.at[src_v.at[m + AHEAD]],
                                     bufs[nslot], gsem[nslot])
            return 0

        lax.fori_loop(0, NCB // NBUF, inner, 0)
        for u in range(NBUF):
            pltpu.make_async_copy(bufs[u], acc_sh.at[dst_v.at[0]],
                                  ssem[u]).wait()
        return 0

    lax.fori_loop(0, NPH, phase, 0)
    plsc.subcore_barrier()
    pltpu.sync_copy(acc_sh.at[pl.ds(s * RPS, RPS)],
                    out_hbm.at[c, pl.ds(s * RPS, RPS)])


def _sc_agg(h, src3, dst3, zeros_blk):
    return pl.kernel(
        _sc_agg_body,
        out_type=jax.ShapeDtypeStruct((NC, NPAD, D), jnp.float32),
        mesh=_mesh(),
        scratch_types=(
            [pltpu.VMEM((NCB, K), jnp.int32),
             pltpu.VMEM((NCB, K), jnp.int32)]
            + [pltpu.VMEM((K, D), jnp.float32)] * NBUF
            + [pltpu.SemaphoreType.DMA] * (2 * NBUF)
            + [pltpu.VMEM_SHARED((NPAD, D), jnp.float32)]
        ),
    )(h, src3, dst3, zeros_blk)


def _tc_prep_body(x_ref, w_ref, degp_ref, dinv_ref, h_ref):
    deg = jnp.sum(degp_ref[...], axis=0) + 1.0
    dinv = lax.rsqrt(deg)
    dinv_ref[...] = dinv[:, None]
    h = jnp.dot(x_ref[...], w_ref[...], preferred_element_type=jnp.float32)
    h_ref[...] = dinv[:, None] * h


def _tc_prep(x_p, w1, deg_parts):
    return pl.pallas_call(
        _tc_prep_body,
        grid=(GRID,),
        in_specs=[
            pl.BlockSpec((RB, D), lambda i: (i, 0)),
            pl.BlockSpec((D, D), lambda i: (0, 0)),
            pl.BlockSpec((NW, RB), lambda i: (0, i)),
        ],
        out_specs=[
            pl.BlockSpec((RB, 1), lambda i: (i, 0)),
            pl.BlockSpec((RB, D), lambda i: (i, 0)),
        ],
        out_shape=[
            jax.ShapeDtypeStruct((NPAD, 1), jnp.float32),
            jax.ShapeDtypeStruct((NPAD, D), jnp.float32),
        ],
    )(x_p, w1, deg_parts)


def _tc_mid_body(p_ref, hprev_ref, dinv_ref, b_ref, w_ref, hn_ref):
    dinv = dinv_ref[...]
    o = dinv * (p_ref[0] + p_ref[1] + hprev_ref[...]) + b_ref[...]
    xn = jnp.maximum(o, 0.0)
    hn = jnp.dot(xn, w_ref[...], preferred_element_type=jnp.float32)
    hn_ref[...] = dinv * hn


def _tc_mid(p, hprev, dinv, b2d, w_next):
    return pl.pallas_call(
        _tc_mid_body,
        grid=(GRID,),
        in_specs=[
            pl.BlockSpec((NC, RB, D), lambda i: (0, i, 0)),
            pl.BlockSpec((RB, D), lambda i: (i, 0)),
            pl.BlockSpec((RB, 1), lambda i: (i, 0)),
            pl.BlockSpec((1, D), lambda i: (0, 0)),
            pl.BlockSpec((D, D), lambda i: (0, 0)),
        ],
        out_specs=pl.BlockSpec((RB, D), lambda i: (i, 0)),
        out_shape=jax.ShapeDtypeStruct((NPAD, D), jnp.float32),
    )(p, hprev, dinv, b2d, w_next)


def _tc_final_body(p_ref, hprev_ref, dinv_ref, b_ref, out_ref):
    out_ref[...] = (dinv_ref[...] * (p_ref[0] + p_ref[1] + hprev_ref[...])
                    + b_ref[...])


def _tc_final(p, hprev, dinv, b2d):
    return pl.pallas_call(
        _tc_final_body,
        grid=(GRID,),
        in_specs=[
            pl.BlockSpec((NC, RB, D), lambda i: (0, i, 0)),
            pl.BlockSpec((RB, D), lambda i: (i, 0)),
            pl.BlockSpec((RB, 1), lambda i: (i, 0)),
            pl.BlockSpec((1, D), lambda i: (0, 0)),
        ],
        out_specs=pl.BlockSpec((RB, D), lambda i: (i, 0)),
        out_shape=jax.ShapeDtypeStruct((NPAD, D), jnp.float32),
    )(p, hprev, dinv, b2d)


def kernel(x, edge_index, W1, b1, W2, b2, W3, b3):
    src = edge_index[0].astype(jnp.int32)
    dst = edge_index[1].astype(jnp.int32)
    pad = jnp.full((EPAD - E,), N, dtype=jnp.int32)
    src3 = jnp.concatenate([src, pad]).reshape(NW, NCHUNK, K)
    dst3 = jnp.concatenate([dst, pad]).reshape(NW, NCHUNK, K)
    dst2d = dst3.reshape(NW, EPW)
    x_p = jnp.concatenate([x, jnp.zeros((NPAD - N, D), jnp.float32)], axis=0)
    zeros_blk = jnp.zeros((RPS, D), jnp.float32)

    deg_parts = _sc_deg(dst2d)
    dinv, h1 = _tc_prep(x_p, W1, deg_parts)
    b1_2, b2_2, b3_2 = b1[None, :], b2[None, :], b3[None, :]

    p1 = _sc_agg(h1, src3, dst3, zeros_blk)
    h2 = _tc_mid(p1, h1, dinv, b1_2, W2)
    p2 = _sc_agg(h2, src3, dst3, zeros_blk)
    h3 = _tc_mid(p2, h2, dinv, b2_2, W3)
    p3 = _sc_agg(h3, src3, dst3, zeros_blk)
    out = _tc_final(p3, h3, dinv, b3_2)
    return out[:N]


# speedup vs baseline: 9.7019x; 1.0780x over previous
import jax
import jax.numpy as jnp
from jax import lax
from jax.experimental import pallas as pl
from jax.experimental.pallas import tpu as pltpu
from jax.experimental.pallas import tpu_sc as plsc

N = 10000
D = 128
E = 320000

NC = 2
NS = 16
NW = NC * NS
K = 64
TOT = 5120
C0 = 240
C1 = 80
BASE1 = NS * C0
NCB = 40
NBUF = 4
AHEAD = 2
EPAD = TOT * K
EPW = EPAD // NW
NPAD = 10240
RPS = NPAD // NS
RB = 640
GRID = NPAD // RB

assert NS * (C0 + C1) == TOT and C0 % NCB == 0 and C1 % NCB == 0


def _mesh():
    return plsc.VectorSubcoreMesh(
        core_axis_name="c", subcore_axis_name="s", num_cores=NC, num_subcores=NS
    )


def _sc_deg_body(dst_hbm, out_hbm, dst_v, deg_v):
    c = lax.axis_index("c")
    s = lax.axis_index("s")
    wid = s * NC + c
    pltpu.sync_copy(dst_hbm.at[wid], dst_v)
    zeros16 = jnp.zeros((16,), jnp.float32)
    ones16 = jnp.ones((16,), jnp.float32)

    def zero_body(i, _):
        deg_v[pl.ds(i * 16, 16)] = zeros16
        return 0

    lax.fori_loop(0, NPAD // 16, zero_body, 0)

    def scat_body(i, _):
        d16 = dst_v[pl.ds(i * 16, 16)]
        plsc.addupdate_scatter(deg_v, [d16], ones16)
        return 0

    lax.fori_loop(0, EPW // 16, scat_body, 0)
    pltpu.sync_copy(deg_v, out_hbm.at[wid])


def _sc_deg(dst2d):
    return pl.kernel(
        _sc_deg_body,
        out_type=jax.ShapeDtypeStruct((NW, NPAD), jnp.float32),
        mesh=_mesh(),
        scratch_types=[
            pltpu.VMEM((EPW,), jnp.int32),
            pltpu.VMEM((NPAD,), jnp.float32),
        ],
        compiler_params=pltpu.CompilerParams(needs_layout_passes=False),
    )(dst2d)


def _sc_agg_body(h_hbm, src_hbm, dst_hbm, zeros_hbm, out_hbm,
                 src_v, dst_v, *rest):
    bufs = rest[0:NBUF]
    gsem = rest[NBUF:2 * NBUF]
    ssem = rest[2 * NBUF:3 * NBUF]
    acc_sh = rest[3 * NBUF]
    c = lax.axis_index("c")
    s = lax.axis_index("s")

    rows = pl.ds(s * RPS, RPS)
    pltpu.sync_copy(zeros_hbm, acc_sh.at[rows])
    plsc.subcore_barrier()

    def do_edges(base, nph):
        def phase(p, _):
            off = pl.multiple_of(base + p * NCB, NCB)
            pltpu.sync_copy(src_hbm.at[pl.ds(off, NCB)], src_v)
            pltpu.sync_copy(dst_hbm.at[pl.ds(off, NCB)], dst_v)

            for u in range(AHEAD):
                pltpu.async_copy(h_hbm.at[src_v.at[u]], bufs[u], gsem[u])

            def inner(i, _):
                for u in range(NBUF):
                    m = i * NBUF + u
                    pltpu.make_async_copy(h_hbm.at[src_v.at[m]], bufs[u],
                                          gsem[u]).wait()
                    pltpu.async_copy(bufs[u], acc_sh.at[dst_v.at[m]],
                                     ssem[u], add=True)
                    nslot = (u + AHEAD) % NBUF

                    @pl.when(m + AHEAD < NCB)
                    def _():
                        @pl.when(m >= NBUF - AHEAD)
                        def _():
                            pltpu.make_async_copy(
                                bufs[nslot], acc_sh.at[dst_v.at[0]],
                                ssem[nslot]).wait()

                        pltpu.async_copy(h_hbm.at[src_v.at[m + AHEAD]],
                                         bufs[nslot], gsem[nslot])
                return 0

            lax.fori_loop(0, NCB // NBUF, inner, 0)
            for u in range(NBUF):
                pltpu.make_async_copy(bufs[u], acc_sh.at[dst_v.at[0]],
                                      ssem[u]).wait()
            return 0

        lax.fori_loop(0, nph, phase, 0)

    @pl.when(c == 0)
    def _():
        do_edges(s * C0, C0 // NCB)

    @pl.when(c == 1)
    def _():
        do_edges(BASE1 + s * C1, C1 // NCB)

    plsc.subcore_barrier()
    pltpu.sync_copy(acc_sh.at[rows], out_hbm.at[c, rows])


def _sc_agg(h, src2, dst2, zeros_blk):
    return pl.kernel(
        _sc_agg_body,
        out_type=jax.ShapeDtypeStruct((NC, NPAD, D), jnp.float32),
        mesh=_mesh(),
        scratch_types=(
            [pltpu.VMEM((NCB, K), jnp.int32),
             pltpu.VMEM((NCB, K), jnp.int32)]
            + [pltpu.VMEM((K, D), jnp.float32)] * NBUF
            + [pltpu.SemaphoreType.DMA] * (2 * NBUF)
            + [pltpu.VMEM_SHARED((NPAD, D), jnp.float32)]
        ),
    )(h, src2, dst2, zeros_blk)


def _tc_prep_body(x_ref, w_ref, degp_ref, dinv_ref, h_ref):
    deg = jnp.sum(degp_ref[...], axis=0) + 1.0
    dinv = lax.rsqrt(deg)
    dinv_ref[...] = dinv[:, None]
    h = jnp.dot(x_ref[...], w_ref[...], preferred_element_type=jnp.float32)
    h_ref[...] = dinv[:, None] * h


def _tc_prep(x_p, w1, deg_parts):
    return pl.pallas_call(
        _tc_prep_body,
        grid=(GRID,),
        in_specs=[
            pl.BlockSpec((RB, D), lambda i: (i, 0)),
            pl.BlockSpec((D, D), lambda i: (0, 0)),
            pl.BlockSpec((NW, RB), lambda i: (0, i)),
        ],
        out_specs=[
            pl.BlockSpec((RB, 1), lambda i: (i, 0)),
            pl.BlockSpec((RB, D), lambda i: (i, 0)),
        ],
        out_shape=[
            jax.ShapeDtypeStruct((NPAD, 1), jnp.float32),
            jax.ShapeDtypeStruct((NPAD, D), jnp.float32),
        ],
    )(x_p, w1, deg_parts)


def _tc_mid_body(p_ref, hprev_ref, dinv_ref, b_ref, w_ref, hn_ref):
    dinv = dinv_ref[...]
    o = dinv * (p_ref[0] + p_ref[1] + hprev_ref[...]) + b_ref[...]
    xn = jnp.maximum(o, 0.0)
    hn = jnp.dot(xn, w_ref[...], preferred_element_type=jnp.float32)
    hn_ref[...] = dinv * hn


def _tc_mid(p, hprev, dinv, b2d, w_next):
    return pl.pallas_call(
        _tc_mid_body,
        grid=(GRID,),
        in_specs=[
            pl.BlockSpec((NC, RB, D), lambda i: (0, i, 0)),
            pl.BlockSpec((RB, D), lambda i: (i, 0)),
            pl.BlockSpec((RB, 1), lambda i: (i, 0)),
            pl.BlockSpec((1, D), lambda i: (0, 0)),
            pl.BlockSpec((D, D), lambda i: (0, 0)),
        ],
        out_specs=pl.BlockSpec((RB, D), lambda i: (i, 0)),
        out_shape=jax.ShapeDtypeStruct((NPAD, D), jnp.float32),
    )(p, hprev, dinv, b2d, w_next)


def _tc_final_body(p_ref, hprev_ref, dinv_ref, b_ref, out_ref):
    out_ref[...] = (dinv_ref[...] * (p_ref[0] + p_ref[1] + hprev_ref[...])
                    + b_ref[...])


def _tc_final(p, hprev, dinv, b2d):
    return pl.pallas_call(
        _tc_final_body,
        grid=(GRID,),
        in_specs=[
            pl.BlockSpec((NC, RB, D), lambda i: (0, i, 0)),
            pl.BlockSpec((RB, D), lambda i: (i, 0)),
            pl.BlockSpec((RB, 1), lambda i: (i, 0)),
            pl.BlockSpec((1, D), lambda i: (0, 0)),
        ],
        out_specs=pl.BlockSpec((RB, D), lambda i: (i, 0)),
        out_shape=jax.ShapeDtypeStruct((NPAD, D), jnp.float32),
    )(p, hprev, dinv, b2d)


def kernel(x, edge_index, W1, b1, W2, b2, W3, b3):
    src = edge_index[0].astype(jnp.int32)
    dst = edge_index[1].astype(jnp.int32)
    pad = jnp.full((EPAD - E,), N, dtype=jnp.int32)
    src2 = jnp.concatenate([src, pad]).reshape(TOT, K)
    dst2 = jnp.concatenate([dst, pad]).reshape(TOT, K)
    dst2d = dst2.reshape(NW, EPW)
    x_p = jnp.concatenate([x, jnp.zeros((NPAD - N, D), jnp.float32)], axis=0)
    zeros_blk = jnp.zeros((RPS, D), jnp.float32)

    deg_parts = _sc_deg(dst2d)
    dinv, h1 = _tc_prep(x_p, W1, deg_parts)
    b1_2, b2_2, b3_2 = b1[None, :], b2[None, :], b3[None, :]

    p1 = _sc_agg(h1, src2, dst2, zeros_blk)
    h2 = _tc_mid(p1, h1, dinv, b1_2, W2)
    p2 = _sc_agg(h2, src2, dst2, zeros_blk)
    h3 = _tc_mid(p2, h2, dinv, b2_2, W3)
    p3 = _sc_agg(h3, src2, dst2, zeros_blk)
    out = _tc_final(p3, h3, dinv, b3_2)
    return out[:N]
